# baseline (speedup 1.0000x reference)
import functools

import jax
import jax.numpy as jnp
from jax.experimental import pallas as pl

_PROBABILITY = 0.1
_NUM_MODALS = 3
_B, _D = 4096, 1024
_BLK = 256


def _zero_row_masks(B):
    rkey = jax.random.key(42)
    k_mask, k_choice = jax.random.split(rkey)
    mask = jax.random.uniform(k_mask, (B,)) <= _PROBABILITY
    choice = jax.random.randint(k_choice, (B,), 0, _NUM_MODALS)
    return [
        (mask & (choice == m)).astype(jnp.float32)[:, None]
        for m in range(_NUM_MODALS)
    ]


def _body(m0, m1, m2, z0, z1, z2, o0, o1, o2):
    o0[...] = jnp.where(z0[...] != 0, jnp.float32(0), m0[...])
    o1[...] = jnp.where(z1[...] != 0, jnp.float32(0), m1[...])
    o2[...] = jnp.where(z2[...] != 0, jnp.float32(0), m2[...])


@jax.jit
def kernel(modal0, modal1, modal2):
    B, D = modal0.shape
    z0, z1, z2 = _zero_row_masks(B)
    row_spec = pl.BlockSpec((_BLK, D), lambda i: (i, 0))
    msk_spec = pl.BlockSpec((_BLK, 1), lambda i: (i, 0))
    out = pl.pallas_call(
        _body,
        grid=(B // _BLK,),
        in_specs=[row_spec, row_spec, row_spec, msk_spec, msk_spec, msk_spec],
        out_specs=[row_spec, row_spec, row_spec],
        out_shape=[jax.ShapeDtypeStruct((B, D), modal0.dtype)] * 3,
    )(modal0, modal1, modal2, z0, z1, z2)
    return tuple(out)


# speedup vs baseline: 1.0388x; 1.0388x over previous
import functools

import jax
import jax.numpy as jnp
from jax.experimental import pallas as pl

_PROBABILITY = 0.1
_NUM_MODALS = 3
_B, _D = 4096, 1024
_BLK = 512


def _zero_row_masks(B):
    rkey = jax.random.key(42)
    k_mask, k_choice = jax.random.split(rkey)
    mask = jax.random.uniform(k_mask, (B,)) <= _PROBABILITY
    choice = jax.random.randint(k_choice, (B,), 0, _NUM_MODALS)
    return [
        (mask & (choice == m)).astype(jnp.float32)[:, None]
        for m in range(_NUM_MODALS)
    ]


def _body(m0, m1, m2, z0, z1, z2, o0, o1, o2):
    o0[...] = jnp.where(z0[...] != 0, jnp.float32(0), m0[...])
    o1[...] = jnp.where(z1[...] != 0, jnp.float32(0), m1[...])
    o2[...] = jnp.where(z2[...] != 0, jnp.float32(0), m2[...])


@jax.jit
def kernel(modal0, modal1, modal2):
    B, D = modal0.shape
    z0, z1, z2 = _zero_row_masks(B)
    row_spec = pl.BlockSpec((_BLK, D), lambda i: (i, 0))
    msk_spec = pl.BlockSpec((_BLK, 1), lambda i: (i, 0))
    out = pl.pallas_call(
        _body,
        grid=(B // _BLK,),
        in_specs=[row_spec, row_spec, row_spec, msk_spec, msk_spec, msk_spec],
        out_specs=[row_spec, row_spec, row_spec],
        out_shape=[jax.ShapeDtypeStruct((B, D), modal0.dtype)] * 3,
    )(modal0, modal1, modal2, z0, z1, z2)
    return tuple(out)
